# baseline (speedup 1.0000x reference)
import jax
import jax.numpy as jnp
import numpy as np
from jax.experimental import pallas as pl
from jax.experimental.pallas import tpu as pltpu

_GAMMA = 4.0
_NB = 10
_P = 512
_F = 32
_DEG = 12
_NPTS = 64

_TNODES = np.cos(np.pi * (np.arange(_NPTS) + 0.5) / _NPTS)
_PINV = np.linalg.pinv(
    np.polynomial.chebyshev.chebvander(_TNODES, _DEG))


def _conv_kernel(params_ref, feat_ref, geom_a_ref, geom_b_ref,
                 wt_ref, out_ref):
    f = feat_ref[0]
    wt = wt_ref[...]
    gb = jnp.dot(f, wt,
                 preferred_element_type=jnp.float32).astype(jnp.bfloat16)

    q2 = jnp.full((_P, _P), params_ref[1], jnp.float32)
    for c in range(3):
        b_row = geom_b_ref[0, c:c + 1, :]
        a_col = geom_a_ref[0, :, c:c + 1]
        diff = b_row - a_col
        wrapped = (diff - jnp.round(diff)) * params_ref[c + 2]
        q2 = q2 + wrapped * wrapped
    inside = q2 <= params_ref[0]
    t = jnp.sqrt(q2) - 1.0
    tm = jnp.where(inside, t, 0.0)
    t2 = tm + tm

    pw = [None] * (_DEG + 1)
    pw[0] = jnp.where(inside, 1.0, 0.0)
    pw[1] = tm
    for j in range(2, _DEG + 1):
        pw[j] = t2 * pw[j - 1] - pw[j - 2]

    acc = jnp.zeros((_P, _F), jnp.float32)
    for j in range(_DEG + 1):
        acc = acc + jnp.dot(pw[j].astype(jnp.bfloat16),
                            gb[:, j * _F:(j + 1) * _F],
                            preferred_element_type=jnp.float32)
    out_ref[0] = acc


def kernel(features, geometry, lattice, W, mu, max_radius):
    B = features.shape[0]
    mu = mu.astype(jnp.float32)
    rmax = jnp.asarray(max_radius, jnp.float32)
    ldiag = jnp.stack([lattice[0, 0], lattice[1, 1], lattice[2, 2]])
    geom_s = geometry.astype(jnp.float32) / ldiag
    geom_t = geom_s.transpose(0, 2, 1)

    dnodes = (jnp.asarray(_TNODES, jnp.float32) + 1.0) * (rmax * 0.5)
    y = jnp.exp(-_GAMMA * (dnodes[:, None] - mu[None, :]) ** 2)
    pcheb = jnp.asarray(_PINV, jnp.float32) @ y
    wt = jnp.einsum('jk,koi->ijo', pcheb, W).reshape(_F, (_DEG + 1) * _F)

    s = 2.0 * ldiag / rmax
    q2max = (2.0 / rmax) ** 2 * (rmax * rmax + 1e-12)
    eps_q2 = (2.0 / rmax) ** 2 * 1e-12
    params = jnp.stack([q2max, eps_q2, s[0], s[1], s[2],
                        jnp.float32(0), jnp.float32(0), jnp.float32(0)])
    return pl.pallas_call(
        _conv_kernel,
        grid=(B,),
        in_specs=[
            pl.BlockSpec(memory_space=pltpu.SMEM),
            pl.BlockSpec((1, _P, _F), lambda z: (z, 0, 0)),
            pl.BlockSpec((1, _P, 3), lambda z: (z, 0, 0)),
            pl.BlockSpec((1, 3, _P), lambda z: (z, 0, 0)),
            pl.BlockSpec((_F, (_DEG + 1) * _F), lambda z: (0, 0)),
        ],
        out_specs=pl.BlockSpec((1, _P, _F), lambda z: (z, 0, 0)),
        out_shape=jax.ShapeDtypeStruct((B, _P, _F), jnp.float32),
    )(params, features, geom_s, geom_t, wt)
